# baseline (speedup 1.0000x reference)
import functools

import jax
import jax.numpy as jnp
from jax import lax
from jax.experimental import pallas as pl
from jax.experimental.pallas import tpu as pltpu
from jax.experimental.pallas import tpu_sc as plsc

NC = 2
NS = 16
CH = 128
CH3 = 64
GROUP = 8


def _deg_body(n_pad, n_chunks, idx_hbm, ones_hbm, zeros_hbm, out_hbm,
              idxbuf, ones_v, zeros_v, deg_sh):
  c = lax.axis_index("c")
  s = lax.axis_index("s")
  w = c * NS + s
  span = n_pad // NS
  pltpu.sync_copy(ones_hbm, ones_v)
  pltpu.sync_copy(zeros_hbm, zeros_v)
  pltpu.sync_copy(zeros_v, deg_sh.at[pl.ds(s * span, span)])
  plsc.subcore_barrier()
  pltpu.sync_copy(idx_hbm.at[w], idxbuf)

  def body(j, carry):
    pltpu.sync_copy(ones_v, deg_sh.at[idxbuf.at[j]], add=True)
    return carry

  lax.fori_loop(0, n_chunks, body, 0)
  plsc.subcore_barrier()
  pltpu.sync_copy(deg_sh.at[pl.ds(s * span, span)], zeros_v)
  pltpu.sync_copy(zeros_v, out_hbm.at[w])


def _make_deg_kernel(n_pad, n_chunks):
  mesh = plsc.VectorSubcoreMesh(core_axis_name="c", subcore_axis_name="s",
                                num_cores=NC, num_subcores=NS)
  return pl.kernel(
      functools.partial(_deg_body, n_pad, n_chunks),
      out_type=jax.ShapeDtypeStruct((NC * NS, n_pad // NS), jnp.float32),
      mesh=mesh,
      scratch_types=[
          pltpu.VMEM((n_chunks, CH), jnp.int32),
          pltpu.VMEM((CH,), jnp.float32),
          pltpu.VMEM((n_pad // NS,), jnp.float32),
          pltpu.VMEM_SHARED((n_pad,), jnp.float32),
      ],
  )


def _mm_body(inc_ref, nh_ref, out_ref):
  out_ref[...] = lax.dot_general(inc_ref[...], nh_ref[...],
                                 (((0,), (0,)), ((), ())),
                                 preferred_element_type=jnp.float32)


def _matmul(curr_inc_t, next_h, n, m, d, blk):
  return pl.pallas_call(
      _mm_body,
      grid=(-(-n // blk),),
      in_specs=[
          pl.BlockSpec((m, blk), lambda i: (0, i)),
          pl.BlockSpec((m, d), lambda i: (0, 0)),
      ],
      out_specs=pl.BlockSpec((blk, d), lambda i: (i, 0)),
      out_shape=jax.ShapeDtypeStruct((n, d), jnp.float32),
  )(curr_inc_t, next_h)


def _scale_body(ch_ref, f_ref, deg_ref, out_ref):
  ns = lax.rsqrt(deg_ref[...] + 1.0)
  out_ref[0] = ch_ref[...] * ns
  out_ref[1] = f_ref[...] * ns


def _scale(curr_h, fused, deg_out_col, n, d, blk):
  return pl.pallas_call(
      _scale_body,
      grid=(n // blk,),
      in_specs=[
          pl.BlockSpec((blk, d), lambda i: (i, 0)),
          pl.BlockSpec((blk, d), lambda i: (i, 0)),
          pl.BlockSpec((blk, 1), lambda i: (i, 0)),
      ],
      out_specs=pl.BlockSpec((2, blk, d), lambda i: (0, i, 0)),
      out_shape=jax.ShapeDtypeStruct((2, n, d), jnp.float32),
  )(curr_h, fused, deg_out_col)


def _agg_body(n_sh, n_chunks, yflat_hbm, idx2_hbm, zeros_hbm, out_hbm,
              iba, ibb, d0, d1, d2, d3, agg_sh, sg0, sg1, sg2, sg3,
              ss0, ss1, ss2, ss3, si0, si1):
  c = lax.axis_index("c")
  s = lax.axis_index("s")
  w = c * NS + s
  base = w * (n_chunks * 2)
  zspan = n_sh // NS
  dbuf = [d0, d1, d2, d3]
  sg = [sg0, sg1, sg2, sg3]
  ss = [ss0, ss1, ss2, ss3]
  ibs = [iba, ibb]
  sis = [si0, si1]
  n_groups = n_chunks // GROUP
  rows = 2 * GROUP

  pltpu.sync_copy(zeros_hbm, d0)
  for k in range(zspan // CH3):
    pltpu.sync_copy(d0, agg_sh.at[pl.ds(s * zspan + k * CH3, CH3)])
  plsc.subcore_barrier()

  def g_row(buf, k):
    return buf.at[2 * (k % GROUP)]

  def s_row(buf, k):
    return buf.at[2 * (k % GROUP) + 1]

  pltpu.sync_copy(idx2_hbm.at[pl.ds(base, rows)], iba)
  pltpu.async_copy(yflat_hbm.at[g_row(iba, 0)], d0, sg0)
  pltpu.async_copy(yflat_hbm.at[g_row(iba, 1)], d1, sg1)

  def gpair(t, carry):
    for gg in range(2):
      g = 2 * t + gg
      cur, nxt = ibs[gg], ibs[1 - gg]
      for k in range(GROUP):
        b = k % 4
        b2 = (b + 2) % 4
        drain_row = s_row(cur, k - 2) if k >= 2 else s_row(nxt, k + 6)
        drain = lambda: pltpu.make_async_copy(
            dbuf[b2], agg_sh.at[drain_row], ss[b2]).wait()
        if gg == 0 and k < 2:
          @pl.when(t > 0)
          def _(drain=drain):
            drain()
        else:
          drain()
        if k < GROUP - 2:
          pltpu.async_copy(yflat_hbm.at[g_row(cur, k + 2)], dbuf[b2], sg[b2])
        else:
          def fire(k=k, b2=b2, g=g, cur=cur, nxt=nxt):
            if k == GROUP - 2:
              pltpu.make_async_copy(
                  idx2_hbm.at[pl.ds(base + (g + 1) * rows, rows)],
                  nxt, sis[1 - gg]).wait()
            pltpu.async_copy(yflat_hbm.at[g_row(nxt, k + 2)], dbuf[b2],
                             sg[b2])
          if gg == 0:
            fire()
          else:
            @pl.when(g + 1 < n_groups)
            def _(fire=fire):
              fire()
        pltpu.make_async_copy(yflat_hbm.at[g_row(cur, k)], dbuf[b], sg[b]
                              ).wait()
        pltpu.async_copy(dbuf[b], agg_sh.at[s_row(cur, k)], ss[b], add=True)
        if k == 2:
          def pref(g=g, nxt=nxt, gg=gg):
            pltpu.async_copy(
                idx2_hbm.at[pl.ds(base + (g + 1) * rows, rows)],
                nxt, sis[1 - gg])
          if gg == 0:
            pref()
          else:
            @pl.when(g + 1 < n_groups)
            def _(pref=pref):
              pref()
    return carry

  lax.fori_loop(0, n_groups // 2, gpair, 0)
  pltpu.make_async_copy(dbuf[2], agg_sh.at[s_row(ibb, GROUP - 2)], ss[2]
                        ).wait()
  pltpu.make_async_copy(dbuf[3], agg_sh.at[s_row(ibb, GROUP - 1)], ss[3]
                        ).wait()
  plsc.subcore_barrier()
  pltpu.sync_copy(agg_sh.at[pl.ds(s * zspan, zspan)],
                  out_hbm.at[c, pl.ds(s * zspan, zspan)])


def _make_agg_kernel(n_sh, n_chunks, d):
  mesh = plsc.VectorSubcoreMesh(core_axis_name="c", subcore_axis_name="s",
                                num_cores=NC, num_subcores=NS)
  return pl.kernel(
      functools.partial(_agg_body, n_sh, n_chunks),
      out_type=jax.ShapeDtypeStruct((NC, n_sh, d), jnp.float32),
      mesh=mesh,
      scratch_types=[
          pltpu.VMEM((2 * GROUP, CH3), jnp.int32),
          pltpu.VMEM((2 * GROUP, CH3), jnp.int32),
          pltpu.VMEM((CH3, d), jnp.float32),
          pltpu.VMEM((CH3, d), jnp.float32),
          pltpu.VMEM((CH3, d), jnp.float32),
          pltpu.VMEM((CH3, d), jnp.float32),
          pltpu.VMEM_SHARED((n_sh, d), jnp.float32),
      ] + [pltpu.SemaphoreType.DMA] * 10,
  )


def _fin_body(agg_ref, y_ref, deg_ref, wc_ref, wf_ref, cw_ref, tw_ref,
              bc_ref, bf_ref, g_ref, b_ref, out_ref):
  nd = lax.rsqrt(deg_ref[...] + 1.0)
  a1 = (agg_ref[0] + y_ref[0]) * nd
  a2 = (agg_ref[1] + y_ref[1]) * nd
  w1 = wc_ref[...] * cw_ref[...]
  w2 = wf_ref[...] * tw_ref[...]

  def mm(x, wgt):
    return lax.dot_general(x, wgt, (((1,), (0,)), ((), ())),
                           preferred_element_type=jnp.float32)

  r = (mm(a1, w1) + mm(a2, w2)
       + bc_ref[...] * cw_ref[...] + bf_ref[...] * tw_ref[...])
  mu = jnp.mean(r, axis=-1, keepdims=True)
  xc = r - mu
  var = jnp.mean(xc * xc, axis=-1, keepdims=True)
  y = xc * lax.rsqrt(var + 1e-5) * g_ref[...] + b_ref[...]
  out_ref[...] = jnp.maximum(y, 0.0)


def _finalize(agg, ycat, deg_in_col, w_conv, w_fusion, cw, tw, bc, bf,
              gamma, beta, n, d, blk):
  row = lambda i: (0, 0)
  return pl.pallas_call(
      _fin_body,
      grid=(n // blk,),
      in_specs=[
          pl.BlockSpec((2, blk, d), lambda i: (0, i, 0)),
          pl.BlockSpec((2, blk, d), lambda i: (0, i, 0)),
          pl.BlockSpec((blk, 1), lambda i: (i, 0)),
          pl.BlockSpec((d, d), row),
          pl.BlockSpec((d, d), row),
          pl.BlockSpec((1, d), row),
          pl.BlockSpec((1, d), row),
          pl.BlockSpec((1, d), row),
          pl.BlockSpec((1, d), row),
          pl.BlockSpec((1, d), row),
          pl.BlockSpec((1, d), row),
      ],
      out_specs=pl.BlockSpec((blk, d), lambda i: (i, 0)),
      out_shape=jax.ShapeDtypeStruct((n, d), jnp.float32),
  )(agg, ycat, deg_in_col, w_conv, w_fusion, cw, tw, bc, bf, gamma, beta)


def kernel(curr_h, next_h, curr_inc, edge_index,
           W_conv, b_conv, W_fusion, b_fusion,
           conv_w, topDown_w, ln_gamma, ln_beta):
  n, d = curr_h.shape
  m = next_h.shape[0]
  e = edge_index.shape[1]

  pt1 = -(-e // (NS * CH))
  pt3 = -(-e // (NS * CH3))
  pt3 = -(-pt3 // (2 * GROUP)) * (2 * GROUP)
  e_pad1 = NS * pt1 * CH
  e_pad3 = NS * pt3 * CH3
  e_pad = max(e_pad1, e_pad3)
  pt1 = e_pad // (NS * CH)
  pt3 = e_pad // (NS * CH3)
  pad = e_pad - e

  n_sh = -(-(n + 8) // (NS * CH)) * (NS * CH)
  n_trash = n_sh - n

  src = edge_index[0].astype(jnp.int32)
  dst = edge_index[1].astype(jnp.int32)
  ar = lax.iota(jnp.int32, pad)
  trash = n + (ar % n_trash)
  readp = ar % 8

  src_tr = jnp.concatenate([src, trash])
  dst_tr = jnp.concatenate([dst, trash])
  src_rd = jnp.concatenate([src, readp])

  idx_deg = jnp.concatenate([src_tr, dst_tr]).reshape(NC * NS, pt1, CH)
  gsrc = jnp.stack([src_rd, src_rd + n]).reshape(NC, NS, pt3, 1, CH3)
  sdst = jnp.broadcast_to(dst_tr.reshape(1, NS, pt3, 1, CH3),
                          (NC, NS, pt3, 1, CH3))
  idx2 = jnp.concatenate([gsrc, sdst], axis=3).reshape(-1, CH3)

  ones_c = jnp.ones((CH,), jnp.float32)
  zeros1 = jnp.zeros((n_sh // NS,), jnp.float32)
  zeros2 = jnp.zeros((CH3, d), jnp.float32)

  deg = _make_deg_kernel(n_sh, pt1)(idx_deg, ones_c, zeros1)
  deg = deg.reshape(NC, n_sh)[:, :n]
  deg_out_col = deg[0].reshape(n, 1)
  deg_in_col = deg[1].reshape(n, 1)

  fused = _matmul(curr_inc.T, next_h, n, m, d, 512)
  ycat = _scale(curr_h, fused, deg_out_col, n, d, 400)

  agg = _make_agg_kernel(n_sh, pt3, d)(
      ycat.reshape(NC * n, d), idx2, zeros2)

  return _finalize(agg, ycat, deg_in_col, W_conv, W_fusion,
                   conv_w.reshape(1, d), topDown_w.reshape(1, d),
                   b_conv.reshape(1, d), b_fusion.reshape(1, d),
                   ln_gamma.reshape(1, d), ln_beta.reshape(1, d),
                   n, d, 400)


# speedup vs baseline: 1.0012x; 1.0012x over previous
import functools

import jax
import jax.numpy as jnp
from jax import lax
from jax.experimental import pallas as pl
from jax.experimental.pallas import tpu as pltpu
from jax.experimental.pallas import tpu_sc as plsc

NC = 2
NS = 16
CH = 128
CH3 = 64
GROUP = 8


def _deg_body(n_pad, n_chunks, idx_hbm, ones_hbm, zeros_hbm, out_hbm,
              idxbuf, ones_v, zeros_v, deg_sh):
  c = lax.axis_index("c")
  s = lax.axis_index("s")
  w = c * NS + s
  span = n_pad // NS
  pltpu.sync_copy(ones_hbm, ones_v)
  pltpu.sync_copy(zeros_hbm, zeros_v)
  pltpu.sync_copy(zeros_v, deg_sh.at[pl.ds(s * span, span)])
  plsc.subcore_barrier()
  pltpu.sync_copy(idx_hbm.at[w], idxbuf)

  def body(j, carry):
    pltpu.sync_copy(ones_v, deg_sh.at[idxbuf.at[j]], add=True)
    return carry

  lax.fori_loop(0, n_chunks, body, 0)
  plsc.subcore_barrier()
  pltpu.sync_copy(deg_sh.at[pl.ds(s * span, span)], zeros_v)
  pltpu.sync_copy(zeros_v, out_hbm.at[w])


def _make_deg_kernel(n_pad, n_chunks):
  mesh = plsc.VectorSubcoreMesh(core_axis_name="c", subcore_axis_name="s",
                                num_cores=NC, num_subcores=NS)
  return pl.kernel(
      functools.partial(_deg_body, n_pad, n_chunks),
      out_type=jax.ShapeDtypeStruct((NC * NS, n_pad // NS), jnp.float32),
      mesh=mesh,
      scratch_types=[
          pltpu.VMEM((n_chunks, CH), jnp.int32),
          pltpu.VMEM((CH,), jnp.float32),
          pltpu.VMEM((n_pad // NS,), jnp.float32),
          pltpu.VMEM_SHARED((n_pad,), jnp.float32),
      ],
  )


def _mm_body(inc_ref, nh_ref, out_ref):
  out_ref[...] = lax.dot_general(inc_ref[...], nh_ref[...],
                                 (((0,), (0,)), ((), ())),
                                 preferred_element_type=jnp.float32)


def _matmul(curr_inc_t, next_h, n, m, d, blk):
  return pl.pallas_call(
      _mm_body,
      grid=(-(-n // blk),),
      in_specs=[
          pl.BlockSpec((m, blk), lambda i: (0, i)),
          pl.BlockSpec((m, d), lambda i: (0, 0)),
      ],
      out_specs=pl.BlockSpec((blk, d), lambda i: (i, 0)),
      out_shape=jax.ShapeDtypeStruct((n, d), jnp.float32),
  )(curr_inc_t, next_h)


def _scale_body(ch_ref, f_ref, deg_ref, out_ref):
  ns = lax.rsqrt(deg_ref[...] + 1.0)
  out_ref[0] = ch_ref[...] * ns
  out_ref[1] = f_ref[...] * ns


def _scale(curr_h, fused, deg_out_col, n, d, blk):
  return pl.pallas_call(
      _scale_body,
      grid=(n // blk,),
      in_specs=[
          pl.BlockSpec((blk, d), lambda i: (i, 0)),
          pl.BlockSpec((blk, d), lambda i: (i, 0)),
          pl.BlockSpec((blk, 1), lambda i: (i, 0)),
      ],
      out_specs=pl.BlockSpec((2, blk, d), lambda i: (0, i, 0)),
      out_shape=jax.ShapeDtypeStruct((2, n, d), jnp.float32),
  )(curr_h, fused, deg_out_col)


def _agg_body(n_sh, n_chunks, yflat_hbm, idx2_hbm, zeros_hbm, out_hbm,
              iba, ibb, d0, d1, d2, d3, agg_sh, sg0, sg1, sg2, sg3,
              ss0, ss1, ss2, ss3, si0, si1):
  c = lax.axis_index("c")
  s = lax.axis_index("s")
  w = c * NS + s
  base = w * (n_chunks * 2)
  zspan = n_sh // NS
  dbuf = [d0, d1, d2, d3]
  sg = [sg0, sg1, sg2, sg3]
  ss = [ss0, ss1, ss2, ss3]
  ibs = [iba, ibb]
  sis = [si0, si1]
  n_groups = n_chunks // GROUP
  rows = 2 * GROUP

  pltpu.sync_copy(zeros_hbm, d0)
  for k in range(zspan // CH3):
    pltpu.sync_copy(d0, agg_sh.at[pl.ds(s * zspan + k * CH3, CH3)])
  plsc.subcore_barrier()

  def g_row(buf, k):
    return buf.at[2 * (k % GROUP)]

  def s_row(buf, k):
    return buf.at[2 * (k % GROUP) + 1]

  pltpu.sync_copy(idx2_hbm.at[pl.ds(base, rows)], iba)
  pltpu.async_copy(yflat_hbm.at[g_row(iba, 0)], d0, sg0)
  pltpu.async_copy(yflat_hbm.at[g_row(iba, 1)], d1, sg1)

  def gpair(t, carry):
    for gg in range(2):
      g = 2 * t + gg
      cur, nxt = ibs[gg], ibs[1 - gg]
      for k in range(GROUP):
        b = k % 4
        b2 = (b + 2) % 4
        drain_row = s_row(cur, k - 2) if k >= 2 else s_row(nxt, k + 6)
        drain = lambda: pltpu.make_async_copy(
            dbuf[b2], agg_sh.at[drain_row], ss[b2]).wait()
        if gg == 0 and k < 2:
          @pl.when(t > 0)
          def _(drain=drain):
            drain()
        else:
          drain()
        if k < GROUP - 2:
          pltpu.async_copy(yflat_hbm.at[g_row(cur, k + 2)], dbuf[b2], sg[b2])
        else:
          def fire(k=k, b2=b2, g=g, cur=cur, nxt=nxt):
            if k == GROUP - 2:
              pltpu.make_async_copy(
                  idx2_hbm.at[pl.ds(base + (g + 1) * rows, rows)],
                  nxt, sis[1 - gg]).wait()
            pltpu.async_copy(yflat_hbm.at[g_row(nxt, k + 2)], dbuf[b2],
                             sg[b2])
          if gg == 0:
            fire()
          else:
            @pl.when(g + 1 < n_groups)
            def _(fire=fire):
              fire()
        pltpu.make_async_copy(yflat_hbm.at[g_row(cur, k)], dbuf[b], sg[b]
                              ).wait()
        pltpu.async_copy(dbuf[b], agg_sh.at[s_row(cur, k)], ss[b], add=True)
        if k == 2:
          def pref(g=g, nxt=nxt, gg=gg):
            pltpu.async_copy(
                idx2_hbm.at[pl.ds(base + (g + 1) * rows, rows)],
                nxt, sis[1 - gg])
          if gg == 0:
            pref()
          else:
            @pl.when(g + 1 < n_groups)
            def _(pref=pref):
              pref()
    return carry

  lax.fori_loop(0, n_groups // 2, gpair, 0)
  pltpu.make_async_copy(dbuf[2], agg_sh.at[s_row(ibb, GROUP - 2)], ss[2]
                        ).wait()
  pltpu.make_async_copy(dbuf[3], agg_sh.at[s_row(ibb, GROUP - 1)], ss[3]
                        ).wait()
  plsc.subcore_barrier()
  pltpu.sync_copy(agg_sh.at[pl.ds(s * zspan, zspan)],
                  out_hbm.at[c, pl.ds(s * zspan, zspan)])


def _make_agg_kernel(n_sh, n_chunks, d):
  mesh = plsc.VectorSubcoreMesh(core_axis_name="c", subcore_axis_name="s",
                                num_cores=NC, num_subcores=NS)
  return pl.kernel(
      functools.partial(_agg_body, n_sh, n_chunks),
      out_type=jax.ShapeDtypeStruct((NC, n_sh, d), jnp.float32),
      mesh=mesh,
      scratch_types=[
          pltpu.VMEM((2 * GROUP, CH3), jnp.int32),
          pltpu.VMEM((2 * GROUP, CH3), jnp.int32),
          pltpu.VMEM((CH3, d), jnp.float32),
          pltpu.VMEM((CH3, d), jnp.float32),
          pltpu.VMEM((CH3, d), jnp.float32),
          pltpu.VMEM((CH3, d), jnp.float32),
          pltpu.VMEM_SHARED((n_sh, d), jnp.float32),
      ] + [pltpu.SemaphoreType.DMA] * 10,
  )


def _fin_body(agg_ref, y_ref, deg_ref, wc_ref, wf_ref, cw_ref, tw_ref,
              bc_ref, bf_ref, g_ref, b_ref, out_ref):
  nd = lax.rsqrt(deg_ref[...] + 1.0)
  a1 = (agg_ref[0] + y_ref[0]) * nd
  a2 = (agg_ref[1] + y_ref[1]) * nd
  w1 = wc_ref[...] * cw_ref[...]
  w2 = wf_ref[...] * tw_ref[...]

  def mm(x, wgt):
    return lax.dot_general(x, wgt, (((1,), (0,)), ((), ())),
                           preferred_element_type=jnp.float32)

  r = (mm(a1, w1) + mm(a2, w2)
       + bc_ref[...] * cw_ref[...] + bf_ref[...] * tw_ref[...])
  mu = jnp.mean(r, axis=-1, keepdims=True)
  xc = r - mu
  var = jnp.mean(xc * xc, axis=-1, keepdims=True)
  y = xc * lax.rsqrt(var + 1e-5) * g_ref[...] + b_ref[...]
  out_ref[...] = jnp.maximum(y, 0.0)


def _finalize(agg, ycat, deg_in_col, w_conv, w_fusion, cw, tw, bc, bf,
              gamma, beta, n, d, blk):
  row = lambda i: (0, 0)
  return pl.pallas_call(
      _fin_body,
      grid=(n // blk,),
      in_specs=[
          pl.BlockSpec((2, blk, d), lambda i: (0, i, 0)),
          pl.BlockSpec((2, blk, d), lambda i: (0, i, 0)),
          pl.BlockSpec((blk, 1), lambda i: (i, 0)),
          pl.BlockSpec((d, d), row),
          pl.BlockSpec((d, d), row),
          pl.BlockSpec((1, d), row),
          pl.BlockSpec((1, d), row),
          pl.BlockSpec((1, d), row),
          pl.BlockSpec((1, d), row),
          pl.BlockSpec((1, d), row),
          pl.BlockSpec((1, d), row),
      ],
      out_specs=pl.BlockSpec((blk, d), lambda i: (i, 0)),
      out_shape=jax.ShapeDtypeStruct((n, d), jnp.float32),
  )(agg, ycat, deg_in_col, w_conv, w_fusion, cw, tw, bc, bf, gamma, beta)


def kernel(curr_h, next_h, curr_inc, edge_index,
           W_conv, b_conv, W_fusion, b_fusion,
           conv_w, topDown_w, ln_gamma, ln_beta):
  n, d = curr_h.shape
  m = next_h.shape[0]
  e = edge_index.shape[1]

  pt1 = -(-e // (NS * CH))
  pt3 = -(-e // (NS * CH3))
  pt3 = -(-pt3 // (2 * GROUP)) * (2 * GROUP)
  e_pad1 = NS * pt1 * CH
  e_pad3 = NS * pt3 * CH3
  e_pad = max(e_pad1, e_pad3)
  pt1 = e_pad // (NS * CH)
  pt3 = e_pad // (NS * CH3)
  pad = e_pad - e

  n_sh = -(-(n + 8) // (NS * CH)) * (NS * CH)
  n_trash = n_sh - n

  src = edge_index[0].astype(jnp.int32)
  dst = edge_index[1].astype(jnp.int32)
  ar = lax.iota(jnp.int32, pad)
  trash = n + (ar % n_trash)
  readp = ar % 8

  src_tr = jnp.concatenate([src, trash])
  dst_tr = jnp.concatenate([dst, trash])
  src_rd = jnp.concatenate([src, readp])

  idx_deg = jnp.concatenate([src_tr, dst_tr]).reshape(NC * NS, pt1, CH)
  gsrc = jnp.stack([src_rd, src_rd + n]).reshape(NC, NS, pt3, 1, CH3)
  sdst = jnp.broadcast_to(dst_tr.reshape(1, NS, pt3, 1, CH3),
                          (NC, NS, pt3, 1, CH3))
  idx2 = jnp.concatenate([gsrc, sdst], axis=3).reshape(-1, CH3)

  ones_c = jnp.ones((CH,), jnp.float32)
  zeros1 = jnp.zeros((n_sh // NS,), jnp.float32)
  zeros2 = jnp.zeros((CH3, d), jnp.float32)

  fused = _matmul(curr_inc.T, next_h, n, m, d, 512)

  deg = _make_deg_kernel(n_sh, pt1)(idx_deg, ones_c, zeros1)
  deg = deg.reshape(NC, n_sh)[:, :n]
  deg_out_col = deg[0].reshape(n, 1)
  deg_in_col = deg[1].reshape(n, 1)

  ycat = _scale(curr_h, fused, deg_out_col, n, d, 400)

  agg = _make_agg_kernel(n_sh, pt3, d)(
      ycat.reshape(NC * n, d), idx2, zeros2)

  return _finalize(agg, ycat, deg_in_col, W_conv, W_fusion,
                   conv_w.reshape(1, d), topDown_w.reshape(1, d),
                   b_conv.reshape(1, d), b_fusion.reshape(1, d),
                   ln_gamma.reshape(1, d), ln_beta.reshape(1, d),
                   n, d, 400)


# speedup vs baseline: 1.0314x; 1.0301x over previous
import functools

import jax
import jax.numpy as jnp
from jax import lax
from jax.experimental import pallas as pl
from jax.experimental.pallas import tpu as pltpu
from jax.experimental.pallas import tpu_sc as plsc

NC = 2
NS = 16
CH = 128
CH3 = 64
GROUP = 8


def _deg_body(n_pad, n_chunks, idx_hbm, ones_hbm, zeros_hbm, out_hbm,
              idxbuf, ones_v, zeros_v, deg_sh):
  c = lax.axis_index("c")
  s = lax.axis_index("s")
  w = c * NS + s
  span = n_pad // NS
  pltpu.sync_copy(ones_hbm, ones_v)
  pltpu.sync_copy(zeros_hbm, zeros_v)
  pltpu.sync_copy(zeros_v, deg_sh.at[pl.ds(s * span, span)])
  plsc.subcore_barrier()
  pltpu.sync_copy(idx_hbm.at[w], idxbuf)

  def body(j, carry):
    pltpu.sync_copy(ones_v, deg_sh.at[idxbuf.at[j]], add=True)
    return carry

  lax.fori_loop(0, n_chunks, body, 0)
  plsc.subcore_barrier()
  pltpu.sync_copy(deg_sh.at[pl.ds(s * span, span)], zeros_v)
  pltpu.sync_copy(zeros_v, out_hbm.at[w])


def _make_deg_kernel(n_pad, n_chunks):
  mesh = plsc.VectorSubcoreMesh(core_axis_name="c", subcore_axis_name="s",
                                num_cores=NC, num_subcores=NS)
  return pl.kernel(
      functools.partial(_deg_body, n_pad, n_chunks),
      out_type=jax.ShapeDtypeStruct((NC * NS, n_pad // NS), jnp.float32),
      mesh=mesh,
      scratch_types=[
          pltpu.VMEM((n_chunks, CH), jnp.int32),
          pltpu.VMEM((CH,), jnp.float32),
          pltpu.VMEM((n_pad // NS,), jnp.float32),
          pltpu.VMEM_SHARED((n_pad,), jnp.float32),
      ],
  )


def _mm_body(inc_ref, nh_ref, ch_ref, deg_ref, out_ref):
  ns = lax.rsqrt(deg_ref[...] + 1.0)
  f = lax.dot_general(inc_ref[...], nh_ref[...], (((0,), (0,)), ((), ())),
                      preferred_element_type=jnp.float32)
  out_ref[0] = ch_ref[...] * ns
  out_ref[1] = f * ns


def _fused_matmul(curr_inc_t, next_h, curr_h, deg_out_col, n, m, d, blk):
  return pl.pallas_call(
      _mm_body,
      grid=(-(-n // blk),),
      in_specs=[
          pl.BlockSpec((m, blk), lambda i: (0, i)),
          pl.BlockSpec((m, d), lambda i: (0, 0)),
          pl.BlockSpec((blk, d), lambda i: (i, 0)),
          pl.BlockSpec((blk, 1), lambda i: (i, 0)),
      ],
      out_specs=pl.BlockSpec((2, blk, d), lambda i: (0, i, 0)),
      out_shape=jax.ShapeDtypeStruct((2, n, d), jnp.float32),
  )(curr_inc_t, next_h, curr_h, deg_out_col)


def _agg_body(n_sh, n_chunks, yflat_hbm, idx2_hbm, zeros_hbm, out_hbm,
              iba, ibb, d0, d1, d2, d3, agg_sh, sg0, sg1, sg2, sg3,
              ss0, ss1, ss2, ss3, si0, si1):
  c = lax.axis_index("c")
  s = lax.axis_index("s")
  w = c * NS + s
  base = w * (n_chunks * 2)
  zspan = n_sh // NS
  dbuf = [d0, d1, d2, d3]
  sg = [sg0, sg1, sg2, sg3]
  ss = [ss0, ss1, ss2, ss3]
  ibs = [iba, ibb]
  sis = [si0, si1]
  n_groups = n_chunks // GROUP
  rows = 2 * GROUP

  pltpu.sync_copy(zeros_hbm, d0)
  for k in range(zspan // CH3):
    pltpu.sync_copy(d0, agg_sh.at[pl.ds(s * zspan + k * CH3, CH3)])
  plsc.subcore_barrier()

  def g_row(buf, k):
    return buf.at[2 * (k % GROUP)]

  def s_row(buf, k):
    return buf.at[2 * (k % GROUP) + 1]

  pltpu.sync_copy(idx2_hbm.at[pl.ds(base, rows)], iba)
  pltpu.async_copy(yflat_hbm.at[g_row(iba, 0)], d0, sg0)
  pltpu.async_copy(yflat_hbm.at[g_row(iba, 1)], d1, sg1)

  def gpair(t, carry):
    for gg in range(2):
      g = 2 * t + gg
      cur, nxt = ibs[gg], ibs[1 - gg]
      for k in range(GROUP):
        b = k % 4
        b2 = (b + 2) % 4
        drain_row = s_row(cur, k - 2) if k >= 2 else s_row(nxt, k + 6)
        drain = lambda: pltpu.make_async_copy(
            dbuf[b2], agg_sh.at[drain_row], ss[b2]).wait()
        if gg == 0 and k < 2:
          @pl.when(t > 0)
          def _(drain=drain):
            drain()
        else:
          drain()
        if k < GROUP - 2:
          pltpu.async_copy(yflat_hbm.at[g_row(cur, k + 2)], dbuf[b2], sg[b2])
        else:
          def fire(k=k, b2=b2, g=g, cur=cur, nxt=nxt):
            if k == GROUP - 2:
              pltpu.make_async_copy(
                  idx2_hbm.at[pl.ds(base + (g + 1) * rows, rows)],
                  nxt, sis[1 - gg]).wait()
            pltpu.async_copy(yflat_hbm.at[g_row(nxt, k + 2)], dbuf[b2],
                             sg[b2])
          if gg == 0:
            fire()
          else:
            @pl.when(g + 1 < n_groups)
            def _(fire=fire):
              fire()
        pltpu.make_async_copy(yflat_hbm.at[g_row(cur, k)], dbuf[b], sg[b]
                              ).wait()
        pltpu.async_copy(dbuf[b], agg_sh.at[s_row(cur, k)], ss[b], add=True)
        if k == 2:
          def pref(g=g, nxt=nxt, gg=gg):
            pltpu.async_copy(
                idx2_hbm.at[pl.ds(base + (g + 1) * rows, rows)],
                nxt, sis[1 - gg])
          if gg == 0:
            pref()
          else:
            @pl.when(g + 1 < n_groups)
            def _(pref=pref):
              pref()
    return carry

  lax.fori_loop(0, n_groups // 2, gpair, 0)
  pltpu.make_async_copy(dbuf[2], agg_sh.at[s_row(ibb, GROUP - 2)], ss[2]
                        ).wait()
  pltpu.make_async_copy(dbuf[3], agg_sh.at[s_row(ibb, GROUP - 1)], ss[3]
                        ).wait()
  plsc.subcore_barrier()
  pltpu.sync_copy(agg_sh.at[pl.ds(s * zspan, zspan)],
                  out_hbm.at[c, pl.ds(s * zspan, zspan)])


def _make_agg_kernel(n_sh, n_chunks, d):
  mesh = plsc.VectorSubcoreMesh(core_axis_name="c", subcore_axis_name="s",
                                num_cores=NC, num_subcores=NS)
  return pl.kernel(
      functools.partial(_agg_body, n_sh, n_chunks),
      out_type=jax.ShapeDtypeStruct((NC, n_sh, d), jnp.float32),
      mesh=mesh,
      scratch_types=[
          pltpu.VMEM((2 * GROUP, CH3), jnp.int32),
          pltpu.VMEM((2 * GROUP, CH3), jnp.int32),
          pltpu.VMEM((CH3, d), jnp.float32),
          pltpu.VMEM((CH3, d), jnp.float32),
          pltpu.VMEM((CH3, d), jnp.float32),
          pltpu.VMEM((CH3, d), jnp.float32),
          pltpu.VMEM_SHARED((n_sh, d), jnp.float32),
      ] + [pltpu.SemaphoreType.DMA] * 10,
  )


def _fin_body(agg_ref, y_ref, deg_ref, wc_ref, wf_ref, cw_ref, tw_ref,
              bc_ref, bf_ref, g_ref, b_ref, out_ref):
  nd = lax.rsqrt(deg_ref[...] + 1.0)
  a1 = (agg_ref[0] + y_ref[0]) * nd
  a2 = (agg_ref[1] + y_ref[1]) * nd
  w1 = wc_ref[...] * cw_ref[...]
  w2 = wf_ref[...] * tw_ref[...]

  def mm(x, wgt):
    return lax.dot_general(x, wgt, (((1,), (0,)), ((), ())),
                           preferred_element_type=jnp.float32)

  r = (mm(a1, w1) + mm(a2, w2)
       + bc_ref[...] * cw_ref[...] + bf_ref[...] * tw_ref[...])
  mu = jnp.mean(r, axis=-1, keepdims=True)
  xc = r - mu
  var = jnp.mean(xc * xc, axis=-1, keepdims=True)
  y = xc * lax.rsqrt(var + 1e-5) * g_ref[...] + b_ref[...]
  out_ref[...] = jnp.maximum(y, 0.0)


def _finalize(agg, ycat, deg_in_col, w_conv, w_fusion, cw, tw, bc, bf,
              gamma, beta, n, d, blk):
  row = lambda i: (0, 0)
  return pl.pallas_call(
      _fin_body,
      grid=(n // blk,),
      in_specs=[
          pl.BlockSpec((2, blk, d), lambda i: (0, i, 0)),
          pl.BlockSpec((2, blk, d), lambda i: (0, i, 0)),
          pl.BlockSpec((blk, 1), lambda i: (i, 0)),
          pl.BlockSpec((d, d), row),
          pl.BlockSpec((d, d), row),
          pl.BlockSpec((1, d), row),
          pl.BlockSpec((1, d), row),
          pl.BlockSpec((1, d), row),
          pl.BlockSpec((1, d), row),
          pl.BlockSpec((1, d), row),
          pl.BlockSpec((1, d), row),
      ],
      out_specs=pl.BlockSpec((blk, d), lambda i: (i, 0)),
      out_shape=jax.ShapeDtypeStruct((n, d), jnp.float32),
  )(agg, ycat, deg_in_col, w_conv, w_fusion, cw, tw, bc, bf, gamma, beta)


def kernel(curr_h, next_h, curr_inc, edge_index,
           W_conv, b_conv, W_fusion, b_fusion,
           conv_w, topDown_w, ln_gamma, ln_beta):
  n, d = curr_h.shape
  m = next_h.shape[0]
  e = edge_index.shape[1]

  pt1 = -(-e // (NS * CH))
  pt3 = -(-e // (NS * CH3))
  pt3 = -(-pt3 // (2 * GROUP)) * (2 * GROUP)
  e_pad1 = NS * pt1 * CH
  e_pad3 = NS * pt3 * CH3
  e_pad = max(e_pad1, e_pad3)
  pt1 = e_pad // (NS * CH)
  pt3 = e_pad // (NS * CH3)
  pad = e_pad - e

  n_sh = -(-(n + 8) // (NS * CH)) * (NS * CH)
  n_trash = n_sh - n

  src = edge_index[0].astype(jnp.int32)
  dst = edge_index[1].astype(jnp.int32)
  ar = lax.iota(jnp.int32, pad)
  trash = n + (ar % n_trash)
  readp = ar % 8

  src_tr = jnp.concatenate([src, trash])
  dst_tr = jnp.concatenate([dst, trash])
  src_rd = jnp.concatenate([src, readp])

  idx_deg = jnp.concatenate([src_tr, dst_tr]).reshape(NC * NS, pt1, CH)
  gsrc = jnp.stack([src_rd, src_rd + n]).reshape(NC, NS, pt3, 1, CH3)
  sdst = jnp.broadcast_to(dst_tr.reshape(1, NS, pt3, 1, CH3),
                          (NC, NS, pt3, 1, CH3))
  idx2 = jnp.concatenate([gsrc, sdst], axis=3).reshape(-1, CH3)

  ones_c = jnp.ones((CH,), jnp.float32)
  zeros1 = jnp.zeros((n_sh // NS,), jnp.float32)
  zeros2 = jnp.zeros((CH3, d), jnp.float32)

  deg = _make_deg_kernel(n_sh, pt1)(idx_deg, ones_c, zeros1)
  deg = deg.reshape(NC, n_sh)[:, :n]
  deg_out_col = deg[0].reshape(n, 1)
  deg_in_col = deg[1].reshape(n, 1)

  ycat = _fused_matmul(curr_inc.T, next_h, curr_h, deg_out_col, n, m, d, 512)

  agg = _make_agg_kernel(n_sh, pt3, d)(
      ycat.reshape(NC * n, d), idx2, zeros2)

  return _finalize(agg, ycat, deg_in_col, W_conv, W_fusion,
                   conv_w.reshape(1, d), topDown_w.reshape(1, d),
                   b_conv.reshape(1, d), b_fusion.reshape(1, d),
                   ln_gamma.reshape(1, d), ln_beta.reshape(1, d),
                   n, d, 400)


# speedup vs baseline: 1.0331x; 1.0017x over previous
import functools

import jax
import jax.numpy as jnp
from jax import lax
from jax.experimental import pallas as pl
from jax.experimental.pallas import tpu as pltpu
from jax.experimental.pallas import tpu_sc as plsc

NC = 2
NS = 16
CH = 128
CH3 = 64
GROUP = 8


def _deg_body(n_pad, n_chunks, idx_hbm, ones_hbm, zeros_hbm, out_hbm,
              idxbuf, ones_v, zeros_v, deg_sh, hsem):
  c = lax.axis_index("c")
  s = lax.axis_index("s")
  w = c * NS + s
  span = n_pad // NS
  pltpu.sync_copy(ones_hbm, ones_v)
  pltpu.sync_copy(zeros_hbm, zeros_v)
  pltpu.sync_copy(zeros_v, deg_sh.at[pl.ds(s * span, span)])
  plsc.subcore_barrier()
  pltpu.sync_copy(idx_hbm.at[w], idxbuf)

  def body(q, carry):
    for k in range(8):
      pltpu.async_copy(ones_v, deg_sh.at[idxbuf.at[8 * q + k]], hsem,
                       add=True)
    for k in range(8):
      pltpu.make_async_copy(ones_v, deg_sh.at[idxbuf.at[8 * q + k]], hsem
                            ).wait()
    return carry

  lax.fori_loop(0, n_chunks // 8, body, 0)
  plsc.subcore_barrier()
  pltpu.sync_copy(deg_sh.at[pl.ds(s * span, span)], zeros_v)
  pltpu.sync_copy(zeros_v, out_hbm.at[w])


def _make_deg_kernel(n_pad, n_chunks):
  mesh = plsc.VectorSubcoreMesh(core_axis_name="c", subcore_axis_name="s",
                                num_cores=NC, num_subcores=NS)
  return pl.kernel(
      functools.partial(_deg_body, n_pad, n_chunks),
      out_type=jax.ShapeDtypeStruct((NC * NS, n_pad // NS), jnp.float32),
      mesh=mesh,
      scratch_types=[
          pltpu.VMEM((n_chunks, CH), jnp.int32),
          pltpu.VMEM((CH,), jnp.float32),
          pltpu.VMEM((n_pad // NS,), jnp.float32),
          pltpu.VMEM_SHARED((n_pad,), jnp.float32),
          pltpu.SemaphoreType.DMA,
      ],
  )


def _mm_body(inc_ref, nh_ref, ch_ref, deg_ref, out_ref):
  ns = lax.rsqrt(deg_ref[...] + 1.0)
  f = lax.dot_general(inc_ref[...], nh_ref[...], (((0,), (0,)), ((), ())),
                      preferred_element_type=jnp.float32)
  out_ref[0] = ch_ref[...] * ns
  out_ref[1] = f * ns


def _fused_matmul(curr_inc_t, next_h, curr_h, deg_out_col, n, m, d, blk):
  return pl.pallas_call(
      _mm_body,
      grid=(-(-n // blk),),
      in_specs=[
          pl.BlockSpec((m, blk), lambda i: (0, i)),
          pl.BlockSpec((m, d), lambda i: (0, 0)),
          pl.BlockSpec((blk, d), lambda i: (i, 0)),
          pl.BlockSpec((blk, 1), lambda i: (i, 0)),
      ],
      out_specs=pl.BlockSpec((2, blk, d), lambda i: (0, i, 0)),
      out_shape=jax.ShapeDtypeStruct((2, n, d), jnp.float32),
  )(curr_inc_t, next_h, curr_h, deg_out_col)


def _agg_body(n_sh, n_chunks, yflat_hbm, idx2_hbm, zeros_hbm, out_hbm,
              iba, ibb, d0, d1, d2, d3, agg_sh, sg0, sg1, sg2, sg3,
              ss0, ss1, ss2, ss3, si0, si1):
  c = lax.axis_index("c")
  s = lax.axis_index("s")
  w = c * NS + s
  base = w * (n_chunks * 2)
  zspan = n_sh // NS
  dbuf = [d0, d1, d2, d3]
  sg = [sg0, sg1, sg2, sg3]
  ss = [ss0, ss1, ss2, ss3]
  ibs = [iba, ibb]
  sis = [si0, si1]
  n_groups = n_chunks // GROUP
  rows = 2 * GROUP

  pltpu.sync_copy(zeros_hbm, d0)
  for k in range(zspan // CH3):
    pltpu.sync_copy(d0, agg_sh.at[pl.ds(s * zspan + k * CH3, CH3)])
  plsc.subcore_barrier()

  def g_row(buf, k):
    return buf.at[2 * (k % GROUP)]

  def s_row(buf, k):
    return buf.at[2 * (k % GROUP) + 1]

  pltpu.sync_copy(idx2_hbm.at[pl.ds(base, rows)], iba)
  pltpu.async_copy(yflat_hbm.at[g_row(iba, 0)], d0, sg0)
  pltpu.async_copy(yflat_hbm.at[g_row(iba, 1)], d1, sg1)

  def gpair(t, carry):
    for gg in range(2):
      g = 2 * t + gg
      cur, nxt = ibs[gg], ibs[1 - gg]
      for k in range(GROUP):
        b = k % 4
        b2 = (b + 2) % 4
        drain_row = s_row(cur, k - 2) if k >= 2 else s_row(nxt, k + 6)
        drain = lambda: pltpu.make_async_copy(
            dbuf[b2], agg_sh.at[drain_row], ss[b2]).wait()
        if gg == 0 and k < 2:
          @pl.when(t > 0)
          def _(drain=drain):
            drain()
        else:
          drain()
        if k < GROUP - 2:
          pltpu.async_copy(yflat_hbm.at[g_row(cur, k + 2)], dbuf[b2], sg[b2])
        else:
          def fire(k=k, b2=b2, g=g, cur=cur, nxt=nxt):
            if k == GROUP - 2:
              pltpu.make_async_copy(
                  idx2_hbm.at[pl.ds(base + (g + 1) * rows, rows)],
                  nxt, sis[1 - gg]).wait()
            pltpu.async_copy(yflat_hbm.at[g_row(nxt, k + 2)], dbuf[b2],
                             sg[b2])
          if gg == 0:
            fire()
          else:
            @pl.when(g + 1 < n_groups)
            def _(fire=fire):
              fire()
        pltpu.make_async_copy(yflat_hbm.at[g_row(cur, k)], dbuf[b], sg[b]
                              ).wait()
        pltpu.async_copy(dbuf[b], agg_sh.at[s_row(cur, k)], ss[b], add=True)
        if k == 2:
          def pref(g=g, nxt=nxt, gg=gg):
            pltpu.async_copy(
                idx2_hbm.at[pl.ds(base + (g + 1) * rows, rows)],
                nxt, sis[1 - gg])
          if gg == 0:
            pref()
          else:
            @pl.when(g + 1 < n_groups)
            def _(pref=pref):
              pref()
    return carry

  lax.fori_loop(0, n_groups // 2, gpair, 0)
  pltpu.make_async_copy(dbuf[2], agg_sh.at[s_row(ibb, GROUP - 2)], ss[2]
                        ).wait()
  pltpu.make_async_copy(dbuf[3], agg_sh.at[s_row(ibb, GROUP - 1)], ss[3]
                        ).wait()
  plsc.subcore_barrier()
  pltpu.sync_copy(agg_sh.at[pl.ds(s * zspan, zspan)],
                  out_hbm.at[c, pl.ds(s * zspan, zspan)])


def _make_agg_kernel(n_sh, n_chunks, d):
  mesh = plsc.VectorSubcoreMesh(core_axis_name="c", subcore_axis_name="s",
                                num_cores=NC, num_subcores=NS)
  return pl.kernel(
      functools.partial(_agg_body, n_sh, n_chunks),
      out_type=jax.ShapeDtypeStruct((NC, n_sh, d), jnp.float32),
      mesh=mesh,
      scratch_types=[
          pltpu.VMEM((2 * GROUP, CH3), jnp.int32),
          pltpu.VMEM((2 * GROUP, CH3), jnp.int32),
          pltpu.VMEM((CH3, d), jnp.float32),
          pltpu.VMEM((CH3, d), jnp.float32),
          pltpu.VMEM((CH3, d), jnp.float32),
          pltpu.VMEM((CH3, d), jnp.float32),
          pltpu.VMEM_SHARED((n_sh, d), jnp.float32),
      ] + [pltpu.SemaphoreType.DMA] * 10,
  )


def _fin_body(agg_ref, y_ref, deg_ref, wc_ref, wf_ref, cw_ref, tw_ref,
              bc_ref, bf_ref, g_ref, b_ref, out_ref):
  nd = lax.rsqrt(deg_ref[...] + 1.0)
  a1 = (agg_ref[0] + y_ref[0]) * nd
  a2 = (agg_ref[1] + y_ref[1]) * nd
  w1 = wc_ref[...] * cw_ref[...]
  w2 = wf_ref[...] * tw_ref[...]

  def mm(x, wgt):
    return lax.dot_general(x, wgt, (((1,), (0,)), ((), ())),
                           preferred_element_type=jnp.float32)

  r = (mm(a1, w1) + mm(a2, w2)
       + bc_ref[...] * cw_ref[...] + bf_ref[...] * tw_ref[...])
  mu = jnp.mean(r, axis=-1, keepdims=True)
  xc = r - mu
  var = jnp.mean(xc * xc, axis=-1, keepdims=True)
  y = xc * lax.rsqrt(var + 1e-5) * g_ref[...] + b_ref[...]
  out_ref[...] = jnp.maximum(y, 0.0)


def _finalize(agg, ycat, deg_in_col, w_conv, w_fusion, cw, tw, bc, bf,
              gamma, beta, n, d, blk):
  row = lambda i: (0, 0)
  return pl.pallas_call(
      _fin_body,
      grid=(n // blk,),
      in_specs=[
          pl.BlockSpec((2, blk, d), lambda i: (0, i, 0)),
          pl.BlockSpec((2, blk, d), lambda i: (0, i, 0)),
          pl.BlockSpec((blk, 1), lambda i: (i, 0)),
          pl.BlockSpec((d, d), row),
          pl.BlockSpec((d, d), row),
          pl.BlockSpec((1, d), row),
          pl.BlockSpec((1, d), row),
          pl.BlockSpec((1, d), row),
          pl.BlockSpec((1, d), row),
          pl.BlockSpec((1, d), row),
          pl.BlockSpec((1, d), row),
      ],
      out_specs=pl.BlockSpec((blk, d), lambda i: (i, 0)),
      out_shape=jax.ShapeDtypeStruct((n, d), jnp.float32),
  )(agg, ycat, deg_in_col, w_conv, w_fusion, cw, tw, bc, bf, gamma, beta)


def kernel(curr_h, next_h, curr_inc, edge_index,
           W_conv, b_conv, W_fusion, b_fusion,
           conv_w, topDown_w, ln_gamma, ln_beta):
  n, d = curr_h.shape
  m = next_h.shape[0]
  e = edge_index.shape[1]

  pt1 = -(-e // (NS * CH))
  pt3 = -(-e // (NS * CH3))
  pt3 = -(-pt3 // (2 * GROUP)) * (2 * GROUP)
  e_pad1 = NS * pt1 * CH
  e_pad3 = NS * pt3 * CH3
  e_pad = max(e_pad1, e_pad3)
  pt1 = e_pad // (NS * CH)
  pt3 = e_pad // (NS * CH3)
  pad = e_pad - e

  n_sh = -(-(n + 8) // (NS * CH)) * (NS * CH)
  n_trash = n_sh - n

  src = edge_index[0].astype(jnp.int32)
  dst = edge_index[1].astype(jnp.int32)
  ar = lax.iota(jnp.int32, pad)
  trash = n + (ar % n_trash)
  readp = ar % 8

  src_tr = jnp.concatenate([src, trash])
  dst_tr = jnp.concatenate([dst, trash])
  src_rd = jnp.concatenate([src, readp])

  idx_deg = jnp.concatenate([src_tr, dst_tr]).reshape(NC * NS, pt1, CH)
  gsrc = jnp.stack([src_rd, src_rd + n]).reshape(NC, NS, pt3, 1, CH3)
  sdst = jnp.broadcast_to(dst_tr.reshape(1, NS, pt3, 1, CH3),
                          (NC, NS, pt3, 1, CH3))
  idx2 = jnp.concatenate([gsrc, sdst], axis=3).reshape(-1, CH3)

  ones_c = jnp.ones((CH,), jnp.float32)
  zeros1 = jnp.zeros((n_sh // NS,), jnp.float32)
  zeros2 = jnp.zeros((CH3, d), jnp.float32)

  deg = _make_deg_kernel(n_sh, pt1)(idx_deg, ones_c, zeros1)
  deg = deg.reshape(NC, n_sh)[:, :n]
  deg_out_col = deg[0].reshape(n, 1)
  deg_in_col = deg[1].reshape(n, 1)

  ycat = _fused_matmul(curr_inc.T, next_h, curr_h, deg_out_col, n, m, d, 512)

  agg = _make_agg_kernel(n_sh, pt3, d)(
      ycat.reshape(NC * n, d), idx2, zeros2)

  return _finalize(agg, ycat, deg_in_col, W_conv, W_fusion,
                   conv_w.reshape(1, d), topDown_w.reshape(1, d),
                   b_conv.reshape(1, d), b_fusion.reshape(1, d),
                   ln_gamma.reshape(1, d), ln_beta.reshape(1, d),
                   n, d, 400)


# speedup vs baseline: 1.0786x; 1.0440x over previous
import functools

import jax
import jax.numpy as jnp
from jax import lax
from jax.experimental import pallas as pl
from jax.experimental.pallas import tpu as pltpu
from jax.experimental.pallas import tpu_sc as plsc

NC = 2
NS = 16
CH = 128
CH3 = 80
GROUP = 8


def _deg_body(n_pad, n_chunks, idx_hbm, ones_hbm, zeros_hbm, out_hbm,
              idxbuf, ones_v, zeros_v, deg_sh, hsem):
  c = lax.axis_index("c")
  s = lax.axis_index("s")
  w = c * NS + s
  span = n_pad // NS
  pltpu.sync_copy(ones_hbm, ones_v)
  pltpu.sync_copy(zeros_hbm, zeros_v)
  pltpu.sync_copy(zeros_v, deg_sh.at[pl.ds(s * span, span)])
  plsc.subcore_barrier()
  pltpu.sync_copy(idx_hbm.at[w], idxbuf)

  def body(q, carry):
    for k in range(8):
      pltpu.async_copy(ones_v, deg_sh.at[idxbuf.at[8 * q + k]], hsem,
                       add=True)
    for k in range(8):
      pltpu.make_async_copy(ones_v, deg_sh.at[idxbuf.at[8 * q + k]], hsem
                            ).wait()
    return carry

  lax.fori_loop(0, n_chunks // 8, body, 0)
  plsc.subcore_barrier()
  pltpu.sync_copy(deg_sh.at[pl.ds(s * span, span)], zeros_v)
  pltpu.sync_copy(zeros_v, out_hbm.at[w])


def _make_deg_kernel(n_pad, n_chunks):
  mesh = plsc.VectorSubcoreMesh(core_axis_name="c", subcore_axis_name="s",
                                num_cores=NC, num_subcores=NS)
  return pl.kernel(
      functools.partial(_deg_body, n_pad, n_chunks),
      out_type=jax.ShapeDtypeStruct((NC * NS, n_pad // NS), jnp.float32),
      mesh=mesh,
      scratch_types=[
          pltpu.VMEM((n_chunks, CH), jnp.int32),
          pltpu.VMEM((CH,), jnp.float32),
          pltpu.VMEM((n_pad // NS,), jnp.float32),
          pltpu.VMEM_SHARED((n_pad,), jnp.float32),
          pltpu.SemaphoreType.DMA,
      ],
  )


def _mm_body(inc_ref, nh_ref, ch_ref, deg_ref, out_ref):
  ns = lax.rsqrt(deg_ref[...] + 1.0)
  f = lax.dot_general(inc_ref[...], nh_ref[...], (((0,), (0,)), ((), ())),
                      preferred_element_type=jnp.float32)
  out_ref[0] = ch_ref[...] * ns
  out_ref[1] = f * ns


def _fused_matmul(curr_inc_t, next_h, curr_h, deg_out_col, n, m, d, blk):
  return pl.pallas_call(
      _mm_body,
      grid=(-(-n // blk),),
      in_specs=[
          pl.BlockSpec((m, blk), lambda i: (0, i)),
          pl.BlockSpec((m, d), lambda i: (0, 0)),
          pl.BlockSpec((blk, d), lambda i: (i, 0)),
          pl.BlockSpec((blk, 1), lambda i: (i, 0)),
      ],
      out_specs=pl.BlockSpec((2, blk, d), lambda i: (0, i, 0)),
      out_shape=jax.ShapeDtypeStruct((2, n, d), jnp.float32),
  )(curr_inc_t, next_h, curr_h, deg_out_col)


def _agg_body(n_sh, n_chunks, yflat_hbm, idx2_hbm, zeros_hbm, out_hbm,
              iba, ibb, d0, d1, d2, d3, agg_sh, sg0, sg1, sg2, sg3,
              ss0, ss1, ss2, ss3, si0, si1):
  c = lax.axis_index("c")
  s = lax.axis_index("s")
  w = c * NS + s
  base = w * (n_chunks * 2)
  zspan = n_sh // NS
  dbuf = [d0, d1, d2, d3]
  sg = [sg0, sg1, sg2, sg3]
  ss = [ss0, ss1, ss2, ss3]
  ibs = [iba, ibb]
  sis = [si0, si1]
  n_groups = n_chunks // GROUP
  rows = 2 * GROUP

  pltpu.sync_copy(zeros_hbm, d0)
  for k in range(zspan // CH3):
    pltpu.sync_copy(d0, agg_sh.at[pl.ds(s * zspan + k * CH3, CH3)])
  plsc.subcore_barrier()

  def g_row(buf, k):
    return buf.at[2 * (k % GROUP)]

  def s_row(buf, k):
    return buf.at[2 * (k % GROUP) + 1]

  pltpu.sync_copy(idx2_hbm.at[pl.ds(base, rows)], iba)
  pltpu.async_copy(yflat_hbm.at[g_row(iba, 0)], d0, sg0)
  pltpu.async_copy(yflat_hbm.at[g_row(iba, 1)], d1, sg1)

  def gpair(t, carry):
    for gg in range(2):
      g = 2 * t + gg
      cur, nxt = ibs[gg], ibs[1 - gg]
      for k in range(GROUP):
        b = k % 4
        b2 = (b + 2) % 4
        drain_row = s_row(cur, k - 2) if k >= 2 else s_row(nxt, k + 6)
        drain = lambda: pltpu.make_async_copy(
            dbuf[b2], agg_sh.at[drain_row], ss[b2]).wait()
        if gg == 0 and k < 2:
          @pl.when(t > 0)
          def _(drain=drain):
            drain()
        else:
          drain()
        if k < GROUP - 2:
          pltpu.async_copy(yflat_hbm.at[g_row(cur, k + 2)], dbuf[b2], sg[b2])
        else:
          def fire(k=k, b2=b2, g=g, cur=cur, nxt=nxt):
            if k == GROUP - 2:
              pltpu.make_async_copy(
                  idx2_hbm.at[pl.ds(base + (g + 1) * rows, rows)],
                  nxt, sis[1 - gg]).wait()
            pltpu.async_copy(yflat_hbm.at[g_row(nxt, k + 2)], dbuf[b2],
                             sg[b2])
          if gg == 0:
            fire()
          else:
            @pl.when(g + 1 < n_groups)
            def _(fire=fire):
              fire()
        pltpu.make_async_copy(yflat_hbm.at[g_row(cur, k)], dbuf[b], sg[b]
                              ).wait()
        pltpu.async_copy(dbuf[b], agg_sh.at[s_row(cur, k)], ss[b], add=True)
        if k == 2:
          def pref(g=g, nxt=nxt, gg=gg):
            pltpu.async_copy(
                idx2_hbm.at[pl.ds(base + (g + 1) * rows, rows)],
                nxt, sis[1 - gg])
          if gg == 0:
            pref()
          else:
            @pl.when(g + 1 < n_groups)
            def _(pref=pref):
              pref()
    return carry

  lax.fori_loop(0, n_groups // 2, gpair, 0)
  pltpu.make_async_copy(dbuf[2], agg_sh.at[s_row(ibb, GROUP - 2)], ss[2]
                        ).wait()
  pltpu.make_async_copy(dbuf[3], agg_sh.at[s_row(ibb, GROUP - 1)], ss[3]
                        ).wait()
  plsc.subcore_barrier()
  pltpu.sync_copy(agg_sh.at[pl.ds(s * zspan, zspan)],
                  out_hbm.at[c, pl.ds(s * zspan, zspan)])


def _make_agg_kernel(n_sh, n_chunks, d):
  mesh = plsc.VectorSubcoreMesh(core_axis_name="c", subcore_axis_name="s",
                                num_cores=NC, num_subcores=NS)
  return pl.kernel(
      functools.partial(_agg_body, n_sh, n_chunks),
      out_type=jax.ShapeDtypeStruct((NC, n_sh, d), jnp.float32),
      mesh=mesh,
      scratch_types=[
          pltpu.VMEM((2 * GROUP, CH3), jnp.int32),
          pltpu.VMEM((2 * GROUP, CH3), jnp.int32),
          pltpu.VMEM((CH3, d), jnp.float32),
          pltpu.VMEM((CH3, d), jnp.float32),
          pltpu.VMEM((CH3, d), jnp.float32),
          pltpu.VMEM((CH3, d), jnp.float32),
          pltpu.VMEM_SHARED((n_sh, d), jnp.float32),
      ] + [pltpu.SemaphoreType.DMA] * 10,
  )


def _fin_body(agg_ref, y_ref, deg_ref, wc_ref, wf_ref, cw_ref, tw_ref,
              bc_ref, bf_ref, g_ref, b_ref, out_ref):
  nd = lax.rsqrt(deg_ref[...] + 1.0)
  a1 = (agg_ref[0] + y_ref[0]) * nd
  a2 = (agg_ref[1] + y_ref[1]) * nd
  w1 = wc_ref[...] * cw_ref[...]
  w2 = wf_ref[...] * tw_ref[...]

  def mm(x, wgt):
    return lax.dot_general(x, wgt, (((1,), (0,)), ((), ())),
                           preferred_element_type=jnp.float32)

  r = (mm(a1, w1) + mm(a2, w2)
       + bc_ref[...] * cw_ref[...] + bf_ref[...] * tw_ref[...])
  mu = jnp.mean(r, axis=-1, keepdims=True)
  xc = r - mu
  var = jnp.mean(xc * xc, axis=-1, keepdims=True)
  y = xc * lax.rsqrt(var + 1e-5) * g_ref[...] + b_ref[...]
  out_ref[...] = jnp.maximum(y, 0.0)


def _finalize(agg, ycat, deg_in_col, w_conv, w_fusion, cw, tw, bc, bf,
              gamma, beta, n, d, blk):
  row = lambda i: (0, 0)
  return pl.pallas_call(
      _fin_body,
      grid=(n // blk,),
      in_specs=[
          pl.BlockSpec((2, blk, d), lambda i: (0, i, 0)),
          pl.BlockSpec((2, blk, d), lambda i: (0, i, 0)),
          pl.BlockSpec((blk, 1), lambda i: (i, 0)),
          pl.BlockSpec((d, d), row),
          pl.BlockSpec((d, d), row),
          pl.BlockSpec((1, d), row),
          pl.BlockSpec((1, d), row),
          pl.BlockSpec((1, d), row),
          pl.BlockSpec((1, d), row),
          pl.BlockSpec((1, d), row),
          pl.BlockSpec((1, d), row),
      ],
      out_specs=pl.BlockSpec((blk, d), lambda i: (i, 0)),
      out_shape=jax.ShapeDtypeStruct((n, d), jnp.float32),
  )(agg, ycat, deg_in_col, w_conv, w_fusion, cw, tw, bc, bf, gamma, beta)


def kernel(curr_h, next_h, curr_inc, edge_index,
           W_conv, b_conv, W_fusion, b_fusion,
           conv_w, topDown_w, ln_gamma, ln_beta):
  n, d = curr_h.shape
  m = next_h.shape[0]
  e = edge_index.shape[1]

  pt1 = -(-e // (NS * CH))
  pt3 = -(-e // (NS * CH3))
  pt3 = -(-pt3 // (2 * GROUP)) * (2 * GROUP)
  e_pad1 = NS * pt1 * CH
  e_pad3 = NS * pt3 * CH3
  e_pad = max(e_pad1, e_pad3)
  pt1 = e_pad // (NS * CH)
  pt3 = e_pad // (NS * CH3)
  pad = e_pad - e

  n_sh = -(-(n + 8) // (NS * CH)) * (NS * CH)
  n_trash = n_sh - n

  src = edge_index[0].astype(jnp.int32)
  dst = edge_index[1].astype(jnp.int32)
  ar = lax.iota(jnp.int32, pad)
  trash = n + (ar % n_trash)
  readp = ar % 8

  src_tr = jnp.concatenate([src, trash])
  dst_tr = jnp.concatenate([dst, trash])
  src_rd = jnp.concatenate([src, readp])

  idx_deg = jnp.concatenate([src_tr, dst_tr]).reshape(NC * NS, pt1, CH)
  gsrc = jnp.stack([src_rd, src_rd + n]).reshape(NC, NS, pt3, 1, CH3)
  sdst = jnp.broadcast_to(dst_tr.reshape(1, NS, pt3, 1, CH3),
                          (NC, NS, pt3, 1, CH3))
  idx2 = jnp.concatenate([gsrc, sdst], axis=3).reshape(-1, CH3)

  ones_c = jnp.ones((CH,), jnp.float32)
  zeros1 = jnp.zeros((n_sh // NS,), jnp.float32)
  zeros2 = jnp.zeros((CH3, d), jnp.float32)

  deg = _make_deg_kernel(n_sh, pt1)(idx_deg, ones_c, zeros1)
  deg = deg.reshape(NC, n_sh)[:, :n]
  deg_out_col = deg[0].reshape(n, 1)
  deg_in_col = deg[1].reshape(n, 1)

  ycat = _fused_matmul(curr_inc.T, next_h, curr_h, deg_out_col, n, m, d, 512)

  agg = _make_agg_kernel(n_sh, pt3, d)(
      ycat.reshape(NC * n, d), idx2, zeros2)

  return _finalize(agg, ycat, deg_in_col, W_conv, W_fusion,
                   conv_w.reshape(1, d), topDown_w.reshape(1, d),
                   b_conv.reshape(1, d), b_fusion.reshape(1, d),
                   ln_gamma.reshape(1, d), ln_beta.reshape(1, d),
                   n, d, 400)
